# baseline (speedup 1.0000x reference)
import functools

import jax
import jax.numpy as jnp
from jax import lax
from jax.experimental import pallas as pl
from jax.experimental.pallas import tpu as pltpu
from jax.experimental.pallas import tpu_sc as plsc

N = 10000
E = 320000
D = 128
H = 128
OUT = 128
G = 128

NC = 2
NS = 16
NW = NC * NS
EP = 327680
EPW = EP // NW
CH = 64
NCH = EPW // CH
NPASS = 4
PASS = NCH // NPASS
NBUF = 4
ACCN = N + 16
RPT = 624
TAIL = N - NS * RPT
ZB = 96
ZT = RPT - 6 * ZB


def _sc_aggregate(h, src3d, dst3d):
    mesh = plsc.VectorSubcoreMesh(core_axis_name="c", subcore_axis_name="s",
                                  num_cores=NC, num_subcores=NS)

    @functools.partial(
        pl.kernel,
        out_type=jax.ShapeDtypeStruct((NC, N, D), jnp.float32),
        mesh=mesh,
        scratch_types=[
            pltpu.VMEM((PASS, CH), jnp.int32),
            pltpu.VMEM((PASS, CH), jnp.int32),
            [pltpu.VMEM((CH, D), jnp.float32)] * NBUF,
            pltpu.VMEM_SHARED((ACCN, D), jnp.float32),
            [pltpu.SemaphoreType.DMA] * NBUF,
        ],
    )
    def agg_kernel(h_hbm, src_hbm, dst_hbm, out_hbm,
                   idxs, idxd, rows, acc, sems):
        c = lax.axis_index("c")
        s = lax.axis_index("s")
        w = c * NS + s
        zv = jnp.zeros((16,), jnp.float32)

        def zrow(i, _):
            def zcol(j, _):
                rows[0][i, pl.ds(j * 16, 16)] = zv
                return 0
            return lax.fori_loop(0, D // 16, zcol, 0)

        lax.fori_loop(0, CH, zrow, 0)
        for j in range(RPT // ZB):
            pltpu.sync_copy(rows[0].at[pl.ds(0, ZB)],
                            acc.at[pl.ds(s * RPT + j * ZB, ZB)])
        pltpu.sync_copy(rows[0].at[pl.ds(0, ZT)],
                        acc.at[pl.ds(s * RPT + 6 * ZB, ZT)])

        @pl.when(s == NS - 1)
        def _():
            pltpu.sync_copy(rows[0].at[pl.ds(0, TAIL)],
                            acc.at[pl.ds(NS * RPT, TAIL)])

        plsc.subcore_barrier()
        for ps in range(NPASS):
            pltpu.sync_copy(src_hbm.at[w, pl.ds(ps * PASS, PASS)], idxs)
            pltpu.sync_copy(dst_hbm.at[w, pl.ds(ps * PASS, PASS)], idxd)
            for k in range(NBUF - 1):
                pltpu.async_copy(h_hbm.at[idxs.at[k]], rows[k], sems[k])

            def body(i, _):
                for k in range(NBUF):
                    g = NBUF * i + k
                    nxt = g + NBUF - 1
                    nb = (k + NBUF - 1) % NBUF

                    @pl.when(nxt < PASS)
                    def _():
                        pltpu.async_copy(h_hbm.at[idxs.at[nxt]],
                                         rows[nb], sems[nb])

                    pltpu.make_async_copy(h_hbm.at[idxs.at[g]],
                                          rows[k], sems[k]).wait()
                    pltpu.sync_copy(rows[k], acc.at[idxd.at[g]], add=True)
                return 0

            lax.fori_loop(0, PASS // NBUF, body, 0)
        plsc.subcore_barrier()
        pltpu.sync_copy(acc.at[pl.ds(s * RPT, RPT)],
                        out_hbm.at[c, pl.ds(s * RPT, RPT)])

        @pl.when(s == NS - 1)
        def _():
            pltpu.sync_copy(acc.at[pl.ds(NS * RPT, TAIL)],
                            out_hbm.at[c, pl.ds(NS * RPT, TAIL)])

    return agg_kernel(h, src3d, dst3d)


ROWS_BLK = 2000
GRID = N // ROWS_BLK


def _mlp_body(h_ref, a0_ref, a1_ref, eps_ref, wa_ref, ba_ref, wb_ref, bb_ref,
              o_ref):
    z = h_ref[...] * (1.0 + eps_ref[0, 0]) + a0_ref[...] + a1_ref[...]
    t = jnp.dot(z, wa_ref[...], preferred_element_type=jnp.float32)
    t = jnp.maximum(t + ba_ref[...], 0.0)
    u = jnp.dot(t, wb_ref[...], preferred_element_type=jnp.float32)
    o_ref[...] = jnp.maximum(u + bb_ref[...], 0.0)


def _tc_mlp(h, agg, eps, wa, ba, wb, bb):
    fo = wb.shape[1]
    return pl.pallas_call(
        _mlp_body,
        grid=(GRID,),
        in_specs=[
            pl.BlockSpec((ROWS_BLK, D), lambda i: (i, 0)),
            pl.BlockSpec((ROWS_BLK, D), lambda i: (i, 0)),
            pl.BlockSpec((ROWS_BLK, D), lambda i: (i, 0)),
            pl.BlockSpec(memory_space=pltpu.SMEM),
            pl.BlockSpec((D, 2 * H), lambda i: (0, 0)),
            pl.BlockSpec((1, 2 * H), lambda i: (0, 0)),
            pl.BlockSpec((2 * H, fo), lambda i: (0, 0)),
            pl.BlockSpec((1, fo), lambda i: (0, 0)),
        ],
        out_specs=pl.BlockSpec((ROWS_BLK, fo), lambda i: (i, 0)),
        out_shape=jax.ShapeDtypeStruct((N, fo), jnp.float32),
    )(h, agg[0], agg[1], eps.reshape(1, 1), wa, ba.reshape(1, -1), wb,
      bb.reshape(1, -1))


def _l3_body(h_ref, a0_ref, a1_ref, eps_ref, wa_ref, ba_ref, wb_ref, bb_ref,
             batch_ref, wf1_ref, bf1_ref, wf2_ref, bf2_ref, o_ref, pool_ref):
    i = pl.program_id(0)
    z = h_ref[...] * (1.0 + eps_ref[0, 0]) + a0_ref[...] + a1_ref[...]
    t = jnp.dot(z, wa_ref[...], preferred_element_type=jnp.float32)
    t = jnp.maximum(t + ba_ref[...], 0.0)
    u = jnp.dot(t, wb_ref[...], preferred_element_type=jnp.float32)
    h3 = jnp.maximum(u + bb_ref[...], 0.0)
    bt = jnp.broadcast_to(batch_ref[0], (G, ROWS_BLK))
    gt = lax.broadcasted_iota(jnp.int32, (G, ROWS_BLK), 0)
    onehot_t = (bt == gt).astype(jnp.float32)
    contrib = jnp.dot(onehot_t, h3, preferred_element_type=jnp.float32)

    @pl.when(i == 0)
    def _():
        pool_ref[...] = contrib

    @pl.when(i > 0)
    def _():
        pool_ref[...] = pool_ref[...] + contrib

    @pl.when(i == pl.num_programs(0) - 1)
    def _():
        p = jnp.dot(pool_ref[...], wf1_ref[...],
                    preferred_element_type=jnp.float32)
        p = jnp.maximum(p + bf1_ref[...], 0.0)
        o_ref[...] = (jnp.sum(p * wf2_ref[...], axis=1, keepdims=True)
                      + bf2_ref[0, 0])


def _tc_l3_pool(h, agg, eps, wa, ba, wb, bb, batch3d, wf1, bf1, wf2, bf2):
    return pl.pallas_call(
        _l3_body,
        grid=(GRID,),
        in_specs=[
            pl.BlockSpec((ROWS_BLK, D), lambda i: (i, 0)),
            pl.BlockSpec((ROWS_BLK, D), lambda i: (i, 0)),
            pl.BlockSpec((ROWS_BLK, D), lambda i: (i, 0)),
            pl.BlockSpec(memory_space=pltpu.SMEM),
            pl.BlockSpec((H, 2 * H), lambda i: (0, 0)),
            pl.BlockSpec((1, 2 * H), lambda i: (0, 0)),
            pl.BlockSpec((2 * H, OUT), lambda i: (0, 0)),
            pl.BlockSpec((1, OUT), lambda i: (0, 0)),
            pl.BlockSpec((1, 1, ROWS_BLK), lambda i: (i, 0, 0)),
            pl.BlockSpec((OUT, OUT), lambda i: (0, 0)),
            pl.BlockSpec((1, OUT), lambda i: (0, 0)),
            pl.BlockSpec((1, OUT), lambda i: (0, 0)),
            pl.BlockSpec(memory_space=pltpu.SMEM),
        ],
        out_specs=pl.BlockSpec((G, 1), lambda i: (0, 0)),
        out_shape=jax.ShapeDtypeStruct((G, 1), jnp.float32),
        scratch_shapes=[pltpu.VMEM((G, OUT), jnp.float32)],
    )(h, agg[0], agg[1], eps.reshape(1, 1), wa, ba.reshape(1, -1), wb,
      bb.reshape(1, -1), batch3d, wf1, bf1.reshape(1, -1),
      wf2.reshape(1, -1), bf2.reshape(1, 1))


def kernel(x, edge_index, batch, eps0, W0a, b0a, W0b, b0b, eps1, W1a, b1a,
           W1b, b1b, eps2, W2a, b2a, W2b, b2b, Wf1, bf1, Wf2, bf2):
    npad = EP - E
    src3d = jnp.concatenate(
        [edge_index[0].astype(jnp.int32), jnp.zeros((npad,), jnp.int32)]
    ).reshape(NW, NCH, CH)
    dst3d = jnp.concatenate(
        [edge_index[1].astype(jnp.int32), jnp.full((npad,), N, jnp.int32)]
    ).reshape(NW, NCH, CH)
    batch3d = batch.astype(jnp.int32).reshape(GRID, 1, ROWS_BLK)

    agg = _sc_aggregate(x, src3d, dst3d)
    h = _tc_mlp(x, agg, eps0, W0a, b0a, W0b, b0b)
    agg = _sc_aggregate(h, src3d, dst3d)
    h = _tc_mlp(h, agg, eps1, W1a, b1a, W1b, b1b)
    agg = _sc_aggregate(h, src3d, dst3d)
    return _tc_l3_pool(h, agg, eps2, W2a, b2a, W2b, b2b, batch3d,
                       Wf1, bf1, Wf2, bf2)


# speedup vs baseline: 3.3768x; 3.3768x over previous
import functools

import jax
import jax.numpy as jnp
from jax import lax
from jax.experimental import pallas as pl
from jax.experimental.pallas import tpu as pltpu
from jax.experimental.pallas import tpu_sc as plsc

N = 10000
E = 320000
D = 128
H = 128
OUT = 128
G = 128

NC = 2
NS = 16
NW = NC * NS
EPW = E // NW
CH = 125
NCH = EPW // CH
HALF = NCH // 2
RPT = 624
TAIL = N - NS * RPT
ZB = 96
ZT = RPT - 6 * ZB


def _sc_aggregate(h, src3d, dst3d):
    mesh = plsc.VectorSubcoreMesh(core_axis_name="c", subcore_axis_name="s",
                                  num_cores=NC, num_subcores=NS)

    @functools.partial(
        pl.kernel,
        out_type=jax.ShapeDtypeStruct((NC, N, D), jnp.float32),
        mesh=mesh,
        scratch_types=[
            pltpu.VMEM((HALF, CH), jnp.int32),
            pltpu.VMEM((HALF, CH), jnp.int32),
            pltpu.VMEM((CH, D), jnp.float32),
            pltpu.VMEM((CH, D), jnp.float32),
            pltpu.VMEM_SHARED((N, D), jnp.float32),
            pltpu.SemaphoreType.DMA,
            pltpu.SemaphoreType.DMA,
        ],
    )
    def agg_kernel(h_hbm, src_hbm, dst_hbm, out_hbm,
                   idxs, idxd, rows0, rows1, acc, sem0, sem1):
        c = lax.axis_index("c")
        s = lax.axis_index("s")
        w = c * NS + s
        zv = jnp.zeros((16,), jnp.float32)

        def zrow(i, _):
            def zcol(j, _):
                rows0[i, pl.ds(j * 16, 16)] = zv
                return 0
            return lax.fori_loop(0, D // 16, zcol, 0)

        lax.fori_loop(0, ZB, zrow, 0)
        for j in range(RPT // ZB):
            pltpu.sync_copy(rows0.at[pl.ds(0, ZB)],
                            acc.at[pl.ds(s * RPT + j * ZB, ZB)])
        pltpu.sync_copy(rows0.at[pl.ds(0, ZT)],
                        acc.at[pl.ds(s * RPT + 6 * ZB, ZT)])

        @pl.when(s == NS - 1)
        def _():
            pltpu.sync_copy(rows0.at[pl.ds(0, TAIL)],
                            acc.at[pl.ds(NS * RPT, TAIL)])

        plsc.subcore_barrier()
        for half in range(2):
            pltpu.sync_copy(src_hbm.at[w, pl.ds(half * HALF, HALF)], idxs)
            pltpu.sync_copy(dst_hbm.at[w, pl.ds(half * HALF, HALF)], idxd)
            pltpu.async_copy(h_hbm.at[idxs.at[0]], rows0, sem0)

            def body(i, _):
                g = 2 * i
                pltpu.async_copy(h_hbm.at[idxs.at[g + 1]], rows1, sem1)
                pltpu.make_async_copy(h_hbm.at[idxs.at[g]], rows0, sem0).wait()
                pltpu.sync_copy(rows0, acc.at[idxd.at[g]], add=True)

                @pl.when(g + 2 < HALF)
                def _():
                    pltpu.async_copy(h_hbm.at[idxs.at[g + 2]], rows0, sem0)

                pltpu.make_async_copy(h_hbm.at[idxs.at[g + 1]], rows1,
                                      sem1).wait()
                pltpu.sync_copy(rows1, acc.at[idxd.at[g + 1]], add=True)
                return 0

            lax.fori_loop(0, HALF // 2, body, 0)
        plsc.subcore_barrier()
        pltpu.sync_copy(acc.at[pl.ds(s * RPT, RPT)],
                        out_hbm.at[c, pl.ds(s * RPT, RPT)])

        @pl.when(s == NS - 1)
        def _():
            pltpu.sync_copy(acc.at[pl.ds(NS * RPT, TAIL)],
                            out_hbm.at[c, pl.ds(NS * RPT, TAIL)])

    return agg_kernel(h, src3d, dst3d)


ROWS_BLK = 1000
GRID = N // ROWS_BLK


def _mlp_body(h_ref, a0_ref, a1_ref, eps_ref, wa_ref, ba_ref, wb_ref, bb_ref,
              o_ref):
    z = h_ref[...] * (1.0 + eps_ref[0, 0]) + a0_ref[...] + a1_ref[...]
    t = jnp.dot(z, wa_ref[...], preferred_element_type=jnp.float32)
    t = jnp.maximum(t + ba_ref[...], 0.0)
    u = jnp.dot(t, wb_ref[...], preferred_element_type=jnp.float32)
    o_ref[...] = jnp.maximum(u + bb_ref[...], 0.0)


def _tc_mlp(h, agg, eps, wa, ba, wb, bb):
    fo = wb.shape[1]
    return pl.pallas_call(
        _mlp_body,
        grid=(GRID,),
        in_specs=[
            pl.BlockSpec((ROWS_BLK, D), lambda i: (i, 0)),
            pl.BlockSpec((ROWS_BLK, D), lambda i: (i, 0)),
            pl.BlockSpec((ROWS_BLK, D), lambda i: (i, 0)),
            pl.BlockSpec(memory_space=pltpu.SMEM),
            pl.BlockSpec((D, 2 * H), lambda i: (0, 0)),
            pl.BlockSpec((1, 2 * H), lambda i: (0, 0)),
            pl.BlockSpec((2 * H, fo), lambda i: (0, 0)),
            pl.BlockSpec((1, fo), lambda i: (0, 0)),
        ],
        out_specs=pl.BlockSpec((ROWS_BLK, fo), lambda i: (i, 0)),
        out_shape=jax.ShapeDtypeStruct((N, fo), jnp.float32),
    )(h, agg[0], agg[1], eps.reshape(1, 1), wa, ba.reshape(1, -1), wb,
      bb.reshape(1, -1))


def _l3_body(h_ref, a0_ref, a1_ref, eps_ref, wa_ref, ba_ref, wb_ref, bb_ref,
             batch_ref, wf1_ref, bf1_ref, wf2_ref, bf2_ref, o_ref, pool_ref):
    i = pl.program_id(0)
    z = h_ref[...] * (1.0 + eps_ref[0, 0]) + a0_ref[...] + a1_ref[...]
    t = jnp.dot(z, wa_ref[...], preferred_element_type=jnp.float32)
    t = jnp.maximum(t + ba_ref[...], 0.0)
    u = jnp.dot(t, wb_ref[...], preferred_element_type=jnp.float32)
    h3 = jnp.maximum(u + bb_ref[...], 0.0)
    bt = jnp.broadcast_to(batch_ref[0], (G, ROWS_BLK))
    gt = lax.broadcasted_iota(jnp.int32, (G, ROWS_BLK), 0)
    onehot_t = (bt == gt).astype(jnp.float32)
    contrib = jnp.dot(onehot_t, h3, preferred_element_type=jnp.float32)

    @pl.when(i == 0)
    def _():
        pool_ref[...] = contrib

    @pl.when(i > 0)
    def _():
        pool_ref[...] = pool_ref[...] + contrib

    @pl.when(i == pl.num_programs(0) - 1)
    def _():
        p = jnp.dot(pool_ref[...], wf1_ref[...],
                    preferred_element_type=jnp.float32)
        p = jnp.maximum(p + bf1_ref[...], 0.0)
        o_ref[...] = (jnp.sum(p * wf2_ref[...], axis=1, keepdims=True)
                      + bf2_ref[0, 0])


def _tc_l3_pool(h, agg, eps, wa, ba, wb, bb, batch3d, wf1, bf1, wf2, bf2):
    return pl.pallas_call(
        _l3_body,
        grid=(GRID,),
        in_specs=[
            pl.BlockSpec((ROWS_BLK, D), lambda i: (i, 0)),
            pl.BlockSpec((ROWS_BLK, D), lambda i: (i, 0)),
            pl.BlockSpec((ROWS_BLK, D), lambda i: (i, 0)),
            pl.BlockSpec(memory_space=pltpu.SMEM),
            pl.BlockSpec((H, 2 * H), lambda i: (0, 0)),
            pl.BlockSpec((1, 2 * H), lambda i: (0, 0)),
            pl.BlockSpec((2 * H, OUT), lambda i: (0, 0)),
            pl.BlockSpec((1, OUT), lambda i: (0, 0)),
            pl.BlockSpec((1, 1, ROWS_BLK), lambda i: (i, 0, 0)),
            pl.BlockSpec((OUT, OUT), lambda i: (0, 0)),
            pl.BlockSpec((1, OUT), lambda i: (0, 0)),
            pl.BlockSpec((1, OUT), lambda i: (0, 0)),
            pl.BlockSpec(memory_space=pltpu.SMEM),
        ],
        out_specs=pl.BlockSpec((G, 1), lambda i: (0, 0)),
        out_shape=jax.ShapeDtypeStruct((G, 1), jnp.float32),
        scratch_shapes=[pltpu.VMEM((G, OUT), jnp.float32)],
    )(h, agg[0], agg[1], eps.reshape(1, 1), wa, ba.reshape(1, -1), wb,
      bb.reshape(1, -1), batch3d, wf1, bf1.reshape(1, -1),
      wf2.reshape(1, -1), bf2.reshape(1, 1))


def kernel(x, edge_index, batch, eps0, W0a, b0a, W0b, b0b, eps1, W1a, b1a,
           W1b, b1b, eps2, W2a, b2a, W2b, b2b, Wf1, bf1, Wf2, bf2):
    src3d = edge_index[0].astype(jnp.int32).reshape(NW, NCH, CH)
    dst3d = edge_index[1].astype(jnp.int32).reshape(NW, NCH, CH)
    batch3d = batch.astype(jnp.int32).reshape(GRID, 1, ROWS_BLK)

    agg = _sc_aggregate(x, src3d, dst3d)
    h = _tc_mlp(x, agg, eps0, W0a, b0a, W0b, b0b)
    agg = _sc_aggregate(h, src3d, dst3d)
    h = _tc_mlp(h, agg, eps1, W1a, b1a, W1b, b1b)
    agg = _sc_aggregate(h, src3d, dst3d)
    return _tc_l3_pool(h, agg, eps2, W2a, b2a, W2b, b2b, batch3d,
                       Wf1, bf1, Wf2, bf2)


# speedup vs baseline: 3.4427x; 1.0195x over previous
import functools

import jax
import jax.numpy as jnp
from jax import lax
from jax.experimental import pallas as pl
from jax.experimental.pallas import tpu as pltpu
from jax.experimental.pallas import tpu_sc as plsc

N = 10000
E = 320000
D = 128
H = 128
OUT = 128
G = 128

NC = 2
NS = 16
NW = NC * NS
EPW = E // NW
CH = 125
NCH = EPW // CH
HALF = NCH // 2
RPT = 624
TAIL = N - NS * RPT
ZB = 96
ZT = RPT - 6 * ZB


def _sc_aggregate(h, src3d, dst3d):
    mesh = plsc.VectorSubcoreMesh(core_axis_name="c", subcore_axis_name="s",
                                  num_cores=NC, num_subcores=NS)

    @functools.partial(
        pl.kernel,
        out_type=jax.ShapeDtypeStruct((NC, N, D), jnp.float32),
        mesh=mesh,
        scratch_types=[
            pltpu.VMEM((HALF, CH), jnp.int32),
            pltpu.VMEM((HALF, CH), jnp.int32),
            pltpu.VMEM((CH, D), jnp.float32),
            pltpu.VMEM((CH, D), jnp.float32),
            pltpu.VMEM_SHARED((N, D), jnp.float32),
            pltpu.SemaphoreType.DMA,
            pltpu.SemaphoreType.DMA,
        ],
    )
    def agg_kernel(h_hbm, src_hbm, dst_hbm, out_hbm,
                   idxs, idxd, rows0, rows1, acc, sem0, sem1):
        c = lax.axis_index("c")
        s = lax.axis_index("s")
        w = c * NS + s
        zv = jnp.zeros((16,), jnp.float32)

        def zrow(i, _):
            def zcol(j, _):
                rows0[i, pl.ds(j * 16, 16)] = zv
                return 0
            return lax.fori_loop(0, D // 16, zcol, 0)

        lax.fori_loop(0, ZB, zrow, 0)
        for j in range(RPT // ZB):
            pltpu.sync_copy(rows0.at[pl.ds(0, ZB)],
                            acc.at[pl.ds(s * RPT + j * ZB, ZB)])
        pltpu.sync_copy(rows0.at[pl.ds(0, ZT)],
                        acc.at[pl.ds(s * RPT + 6 * ZB, ZT)])

        @pl.when(s == NS - 1)
        def _():
            pltpu.sync_copy(rows0.at[pl.ds(0, TAIL)],
                            acc.at[pl.ds(NS * RPT, TAIL)])

        plsc.subcore_barrier()
        for half in range(2):
            pltpu.sync_copy(src_hbm.at[w, pl.ds(half * HALF, HALF)], idxs)
            pltpu.sync_copy(dst_hbm.at[w, pl.ds(half * HALF, HALF)], idxd)
            pltpu.async_copy(h_hbm.at[idxs.at[0]], rows0, sem0)

            def body(i, _):
                g = 2 * i
                pltpu.async_copy(h_hbm.at[idxs.at[g + 1]], rows1, sem1)
                pltpu.make_async_copy(h_hbm.at[idxs.at[g]], rows0, sem0).wait()
                pltpu.sync_copy(rows0, acc.at[idxd.at[g]], add=True)

                @pl.when(g + 2 < HALF)
                def _():
                    pltpu.async_copy(h_hbm.at[idxs.at[g + 2]], rows0, sem0)

                pltpu.make_async_copy(h_hbm.at[idxs.at[g + 1]], rows1,
                                      sem1).wait()
                pltpu.sync_copy(rows1, acc.at[idxd.at[g + 1]], add=True)
                return 0

            lax.fori_loop(0, HALF // 2, body, 0)
        plsc.subcore_barrier()
        pltpu.sync_copy(acc.at[pl.ds(s * RPT, RPT)],
                        out_hbm.at[c, pl.ds(s * RPT, RPT)])

        @pl.when(s == NS - 1)
        def _():
            pltpu.sync_copy(acc.at[pl.ds(NS * RPT, TAIL)],
                            out_hbm.at[c, pl.ds(NS * RPT, TAIL)])

    return agg_kernel(h, src3d, dst3d)


ROWS_BLK = 2000
GRID = N // ROWS_BLK


def _mlp_body(h_ref, a0_ref, a1_ref, eps_ref, wa_ref, ba_ref, wb_ref, bb_ref,
              o_ref):
    z = h_ref[...] * (1.0 + eps_ref[0, 0]) + a0_ref[...] + a1_ref[...]
    t = jnp.dot(z, wa_ref[...], preferred_element_type=jnp.float32)
    t = jnp.maximum(t + ba_ref[...], 0.0)
    u = jnp.dot(t, wb_ref[...], preferred_element_type=jnp.float32)
    o_ref[...] = jnp.maximum(u + bb_ref[...], 0.0)


def _tc_mlp(h, agg, eps, wa, ba, wb, bb):
    fo = wb.shape[1]
    return pl.pallas_call(
        _mlp_body,
        grid=(GRID,),
        in_specs=[
            pl.BlockSpec((ROWS_BLK, D), lambda i: (i, 0)),
            pl.BlockSpec((ROWS_BLK, D), lambda i: (i, 0)),
            pl.BlockSpec((ROWS_BLK, D), lambda i: (i, 0)),
            pl.BlockSpec(memory_space=pltpu.SMEM),
            pl.BlockSpec((D, 2 * H), lambda i: (0, 0)),
            pl.BlockSpec((1, 2 * H), lambda i: (0, 0)),
            pl.BlockSpec((2 * H, fo), lambda i: (0, 0)),
            pl.BlockSpec((1, fo), lambda i: (0, 0)),
        ],
        out_specs=pl.BlockSpec((ROWS_BLK, fo), lambda i: (i, 0)),
        out_shape=jax.ShapeDtypeStruct((N, fo), jnp.float32),
    )(h, agg[0], agg[1], eps.reshape(1, 1), wa, ba.reshape(1, -1), wb,
      bb.reshape(1, -1))


def _l3_body(h_ref, a0_ref, a1_ref, eps_ref, wa_ref, ba_ref, wb_ref, bb_ref,
             batch_ref, wf1_ref, bf1_ref, wf2_ref, bf2_ref, o_ref, pool_ref):
    i = pl.program_id(0)
    z = h_ref[...] * (1.0 + eps_ref[0, 0]) + a0_ref[...] + a1_ref[...]
    t = jnp.dot(z, wa_ref[...], preferred_element_type=jnp.float32)
    t = jnp.maximum(t + ba_ref[...], 0.0)
    u = jnp.dot(t, wb_ref[...], preferred_element_type=jnp.float32)
    h3 = jnp.maximum(u + bb_ref[...], 0.0)
    bt = jnp.broadcast_to(batch_ref[0], (G, ROWS_BLK))
    gt = lax.broadcasted_iota(jnp.int32, (G, ROWS_BLK), 0)
    onehot_t = (bt == gt).astype(jnp.float32)
    contrib = jnp.dot(onehot_t, h3, preferred_element_type=jnp.float32)

    @pl.when(i == 0)
    def _():
        pool_ref[...] = contrib

    @pl.when(i > 0)
    def _():
        pool_ref[...] = pool_ref[...] + contrib

    @pl.when(i == pl.num_programs(0) - 1)
    def _():
        p = jnp.dot(pool_ref[...], wf1_ref[...],
                    preferred_element_type=jnp.float32)
        p = jnp.maximum(p + bf1_ref[...], 0.0)
        o_ref[...] = (jnp.sum(p * wf2_ref[...], axis=1, keepdims=True)
                      + bf2_ref[0, 0])


def _tc_l3_pool(h, agg, eps, wa, ba, wb, bb, batch3d, wf1, bf1, wf2, bf2):
    return pl.pallas_call(
        _l3_body,
        grid=(GRID,),
        in_specs=[
            pl.BlockSpec((ROWS_BLK, D), lambda i: (i, 0)),
            pl.BlockSpec((ROWS_BLK, D), lambda i: (i, 0)),
            pl.BlockSpec((ROWS_BLK, D), lambda i: (i, 0)),
            pl.BlockSpec(memory_space=pltpu.SMEM),
            pl.BlockSpec((H, 2 * H), lambda i: (0, 0)),
            pl.BlockSpec((1, 2 * H), lambda i: (0, 0)),
            pl.BlockSpec((2 * H, OUT), lambda i: (0, 0)),
            pl.BlockSpec((1, OUT), lambda i: (0, 0)),
            pl.BlockSpec((1, 1, ROWS_BLK), lambda i: (i, 0, 0)),
            pl.BlockSpec((OUT, OUT), lambda i: (0, 0)),
            pl.BlockSpec((1, OUT), lambda i: (0, 0)),
            pl.BlockSpec((1, OUT), lambda i: (0, 0)),
            pl.BlockSpec(memory_space=pltpu.SMEM),
        ],
        out_specs=pl.BlockSpec((G, 1), lambda i: (0, 0)),
        out_shape=jax.ShapeDtypeStruct((G, 1), jnp.float32),
        scratch_shapes=[pltpu.VMEM((G, OUT), jnp.float32)],
    )(h, agg[0], agg[1], eps.reshape(1, 1), wa, ba.reshape(1, -1), wb,
      bb.reshape(1, -1), batch3d, wf1, bf1.reshape(1, -1),
      wf2.reshape(1, -1), bf2.reshape(1, 1))


def kernel(x, edge_index, batch, eps0, W0a, b0a, W0b, b0b, eps1, W1a, b1a,
           W1b, b1b, eps2, W2a, b2a, W2b, b2b, Wf1, bf1, Wf2, bf2):
    src3d = edge_index[0].astype(jnp.int32).reshape(NW, NCH, CH)
    dst3d = edge_index[1].astype(jnp.int32).reshape(NW, NCH, CH)
    batch3d = batch.astype(jnp.int32).reshape(GRID, 1, ROWS_BLK)

    agg = _sc_aggregate(x, src3d, dst3d)
    h = _tc_mlp(x, agg, eps0, W0a, b0a, W0b, b0b)
    agg = _sc_aggregate(h, src3d, dst3d)
    h = _tc_mlp(h, agg, eps1, W1a, b1a, W1b, b1b)
    agg = _sc_aggregate(h, src3d, dst3d)
    return _tc_l3_pool(h, agg, eps2, W2a, b2a, W2b, b2b, batch3d,
                       Wf1, bf1, Wf2, bf2)
